# baseline (speedup 1.0000x reference)
import functools

import jax
import jax.numpy as jnp
from jax import lax
from jax.experimental import pallas as pl
from jax.experimental.pallas import tpu as pltpu
from jax.experimental.pallas import tpu_sc as plsc

NUM_TYPES = 1000000
CHANNELS = 64
B = 16384
L = 200

NC = 2
NS = 16
NW = NC * NS

BB = 128
NBLK = B // BB
BLK_PER_W = NBLK // NW


def _sc_gather(types2d, table):
    mesh = plsc.VectorSubcoreMesh(core_axis_name="c", subcore_axis_name="s")

    @functools.partial(
        pl.kernel,
        mesh=mesh,
        out_type=jax.ShapeDtypeStruct((L, 8, NBLK, 8, BB), jnp.float32),
        scratch_types=[
            pltpu.VMEM((BB, L), jnp.int32),
            pltpu.VMEM((L, BB), jnp.int32),
            pltpu.VMEM((2, BB, CHANNELS), jnp.float32),
            pltpu.VMEM((2, CHANNELS, BB + 1), jnp.float32),
            pltpu.SemaphoreType.DMA,
            pltpu.SemaphoreType.DMA,
            pltpu.SemaphoreType.DMA,
        ],
        compiler_params=pltpu.CompilerParams(
            use_tc_tiling_on_sc=False, needs_layout_passes=False
        ),
    )
    def run(idx_hbm, table_hbm, out_hbm, ti_v, tidx_v, rows_v, tr_v,
            sem_g, sem_o0, sem_o1):
        wid = lax.axis_index("s") * NC + lax.axis_index("c")
        sems_o = (sem_o0, sem_o1)
        iota = lax.iota(jnp.int32, 16)

        def fire_gather(l, buf):
            pltpu.async_copy(
                table_hbm.at[tidx_v.at[l]], rows_v.at[buf], sem_g
            )

        def wait_gather(l, buf):
            pltpu.make_async_copy(
                table_hbm.at[tidx_v.at[l]], rows_v.at[buf], sem_g
            ).wait()

        def fire_out(l, blk, buf):
            for ch in range(8):
                pltpu.async_copy(
                    tr_v.at[buf, pl.ds(ch * 8, 8), pl.ds(0, BB)],
                    out_hbm.at[l, ch, blk],
                    sems_o[buf],
                )

        def wait_out(l, blk, buf):
            for ch in range(8):
                pltpu.make_async_copy(
                    tr_v.at[buf, pl.ds(ch * 8, 8), pl.ds(0, BB)],
                    out_hbm.at[l, ch, blk],
                    sems_o[buf],
                ).wait()

        def transpose_rows(buf):
            def bloop(b, c2):
                bfull = jnp.full((16,), b, jnp.int32)
                for cg in range(4):
                    v = rows_v[buf, b, pl.ds(cg * 16, 16)]
                    plsc.store_scatter(
                        tr_v.at[buf], [cg * 16 + iota, bfull], v
                    )
                return c2

            lax.fori_loop(0, BB, bloop, 0)

        def block_body(k, carry):
            blk = wid * BLK_PER_W + k
            pltpu.sync_copy(idx_hbm.at[pl.ds(blk * BB, BB)], ti_v)

            def tloop(l, c2):
                lfull = jnp.full((16,), l, jnp.int32)
                for bg in range(8):
                    v = plsc.load_gather(ti_v, [bg * 16 + iota, lfull])
                    tidx_v[l, pl.ds(bg * 16, 16)] = v
                return c2

            lax.fori_loop(0, L, tloop, 0)

            fire_gather(0, 0)

            def lstep(l, c2):
                def body_for(buf):
                    nbuf = 1 - buf
                    wait_gather(l, buf)

                    @pl.when(l + 1 < L)
                    def _():
                        fire_gather(l + 1, nbuf)

                    @pl.when(l >= 2)
                    def _():
                        wait_out(l - 2, blk, buf)

                    transpose_rows(buf)
                    fire_out(l, blk, buf)

                lax.cond(l % 2 == 0, lambda: body_for(0), lambda: body_for(1))
                return c2

            lax.fori_loop(0, L, lstep, 0)
            wait_out(L - 2, blk, 0)
            wait_out(L - 1, blk, 1)
            return carry

        lax.fori_loop(0, BLK_PER_W, block_body, 0)

    return run(types2d, table)


def kernel(types, table):
    out6 = _sc_gather(types, table)
    return out6.transpose(2, 4, 0, 1, 3).reshape(B, L, CHANNELS)


# speedup vs baseline: 1.0408x; 1.0408x over previous
import functools

import jax
import jax.numpy as jnp
from jax import lax
from jax.experimental import pallas as pl
from jax.experimental.pallas import tpu as pltpu
from jax.experimental.pallas import tpu_sc as plsc

NUM_TYPES = 1000000
CHANNELS = 64
B = 16384
L = 200

NC = 2
NS = 16
NW = NC * NS

BB = 128
NBLK = B // BB
BLK_PER_W = NBLK // NW


def _sc_gather(types2d, table):
    mesh = plsc.VectorSubcoreMesh(core_axis_name="c", subcore_axis_name="s")

    @functools.partial(
        pl.kernel,
        mesh=mesh,
        out_type=jax.ShapeDtypeStruct((L, 8, NBLK, 8, BB), jnp.float32),
        scratch_types=[
            pltpu.VMEM((BB, L), jnp.int32),
            pltpu.VMEM((L, BB), jnp.int32),
            pltpu.VMEM((2, BB, CHANNELS), jnp.float32),
            pltpu.VMEM((2, CHANNELS, BB + 1), jnp.float32),
            pltpu.SemaphoreType.DMA,
            pltpu.SemaphoreType.DMA,
            pltpu.SemaphoreType.DMA,
        ],
        compiler_params=pltpu.CompilerParams(
            use_tc_tiling_on_sc=False, needs_layout_passes=False
        ),
    )
    def run(idx_hbm, table_hbm, out_hbm, ti_v, tidx_v, rows_v, tr_v,
            sem_g, sem_o0, sem_o1):
        wid = lax.axis_index("s") * NC + lax.axis_index("c")
        sems_o = (sem_o0, sem_o1)
        iota = lax.iota(jnp.int32, 16)

        def fire_gather(l, buf):
            pltpu.async_copy(
                table_hbm.at[tidx_v.at[l]], rows_v.at[buf], sem_g
            )

        def wait_gather(l, buf):
            pltpu.make_async_copy(
                table_hbm.at[tidx_v.at[l]], rows_v.at[buf], sem_g
            ).wait()

        def fire_out(l, blk, buf):
            for ch in range(8):
                pltpu.async_copy(
                    tr_v.at[buf, pl.ds(ch * 8, 8), pl.ds(0, BB)],
                    out_hbm.at[l, ch, blk],
                    sems_o[buf],
                )

        def wait_out(l, blk, buf):
            for ch in range(8):
                pltpu.make_async_copy(
                    tr_v.at[buf, pl.ds(ch * 8, 8), pl.ds(0, BB)],
                    out_hbm.at[l, ch, blk],
                    sems_o[buf],
                ).wait()

        def transpose_rows(buf):
            def bloop(b4, c2):
                for db in range(4):
                    b = b4 * 4 + db
                    bfull = jnp.full((16,), b, jnp.int32)
                    for cg in range(4):
                        v = rows_v[buf, b, pl.ds(cg * 16, 16)]
                        plsc.store_scatter(
                            tr_v.at[buf], [cg * 16 + iota, bfull], v
                        )
                return c2

            lax.fori_loop(0, BB // 4, bloop, 0)

        def block_body(k, carry):
            blk = wid * BLK_PER_W + k
            pltpu.sync_copy(idx_hbm.at[pl.ds(blk * BB, BB)], ti_v)

            def tloop(l, c2):
                lfull = jnp.full((16,), l, jnp.int32)
                for bg in range(8):
                    v = plsc.load_gather(ti_v, [bg * 16 + iota, lfull])
                    tidx_v[l, pl.ds(bg * 16, 16)] = v
                return c2

            lax.fori_loop(0, L, tloop, 0)

            fire_gather(0, 0)

            def lstep(l, c2):
                def body_for(buf):
                    nbuf = 1 - buf
                    wait_gather(l, buf)

                    @pl.when(l + 1 < L)
                    def _():
                        fire_gather(l + 1, nbuf)

                    @pl.when(l >= 2)
                    def _():
                        wait_out(l - 2, blk, buf)

                    transpose_rows(buf)
                    fire_out(l, blk, buf)

                lax.cond(l % 2 == 0, lambda: body_for(0), lambda: body_for(1))
                return c2

            lax.fori_loop(0, L, lstep, 0)
            wait_out(L - 2, blk, 0)
            wait_out(L - 1, blk, 1)
            return carry

        lax.fori_loop(0, BLK_PER_W, block_body, 0)

    return run(types2d, table)


def kernel(types, table):
    out6 = _sc_gather(types, table)
    return out6.transpose(2, 4, 0, 1, 3).reshape(B, L, CHANNELS)


# speedup vs baseline: 1.2003x; 1.1533x over previous
import functools

import jax
import jax.numpy as jnp
from jax import lax
from jax.experimental import pallas as pl
from jax.experimental.pallas import tpu as pltpu
from jax.experimental.pallas import tpu_sc as plsc

NUM_TYPES = 1000000
CHANNELS = 64
B = 16384
L = 200

NC = 2
NS = 16
NW = NC * NS

GW = 128
N_FLAT = B * L
K = 5
KI = K * GW
PER_W = N_FLAT // NW
NCH = PER_W // KI


def _sc_gather(types_flat, table):
    mesh = plsc.VectorSubcoreMesh(core_axis_name="c", subcore_axis_name="s")

    @functools.partial(
        pl.kernel,
        mesh=mesh,
        out_type=jax.ShapeDtypeStruct((N_FLAT, 2 * CHANNELS), jnp.float32),
        scratch_types=[
            pltpu.VMEM((2, KI), jnp.int32),
            pltpu.VMEM((2, KI, CHANNELS), jnp.float32),
            pltpu.SemaphoreType.DMA,
            pltpu.SemaphoreType.DMA,
            pltpu.SemaphoreType.DMA,
        ],
        compiler_params=pltpu.CompilerParams(use_tc_tiling_on_sc=False),
    )
    def run(idx_hbm, table_hbm, out_hbm, idx_v, rows_v, sem_g, sem_o0, sem_o1):
        wid = lax.axis_index("s") * NC + lax.axis_index("c")
        w_base = wid * PER_W
        sems_o = (sem_o0, sem_o1)

        def fire_chunk(c, buf):
            base = w_base + c * KI
            pltpu.sync_copy(idx_hbm.at[pl.ds(base, KI)], idx_v.at[buf])
            for j in range(K):
                pltpu.async_copy(
                    table_hbm.at[idx_v.at[buf, pl.ds(j * GW, GW)]],
                    rows_v.at[buf, pl.ds(j * GW, GW)],
                    sem_g,
                )

        def drain_gathers(c, buf):
            for j in range(K):
                pltpu.make_async_copy(
                    table_hbm.at[idx_v.at[buf, pl.ds(j * GW, GW)]],
                    rows_v.at[buf, pl.ds(j * GW, GW)],
                    sem_g,
                ).wait()

        def out_copy(c, buf):
            base = w_base + c * KI
            pltpu.async_copy(
                rows_v.at[buf],
                out_hbm.at[pl.ds(base, KI), pl.ds(0, CHANNELS)],
                sems_o[buf],
            )

        def wait_out(c, buf):
            base = w_base + c * KI
            pltpu.make_async_copy(
                rows_v.at[buf],
                out_hbm.at[pl.ds(base, KI), pl.ds(0, CHANNELS)],
                sems_o[buf],
            ).wait()

        fire_chunk(0, 0)

        def step(c, carry):
            def body_for(buf):
                nbuf = 1 - buf
                drain_gathers(c, buf)
                out_copy(c, buf)

                @pl.when(c + 1 < NCH)
                def _():
                    @pl.when(c >= 1)
                    def _():
                        wait_out(c - 1, nbuf)

                    fire_chunk(c + 1, nbuf)

            lax.cond(c % 2 == 0, lambda: body_for(0), lambda: body_for(1))
            return carry

        lax.fori_loop(0, NCH, step, 0)
        wait_out(NCH - 2, (NCH - 2) % 2)
        wait_out(NCH - 1, (NCH - 1) % 2)

    return run(types_flat, table)


def kernel(types, table):
    out = _sc_gather(types.reshape(N_FLAT), table)
    return out[:, :CHANNELS].reshape(B, L, CHANNELS)


# speedup vs baseline: 1.2168x; 1.0137x over previous
import functools

import jax
import jax.numpy as jnp
from jax import lax
from jax.experimental import pallas as pl
from jax.experimental.pallas import tpu as pltpu
from jax.experimental.pallas import tpu_sc as plsc

NUM_TYPES = 1000000
CHANNELS = 64
B = 16384
L = 200

NC = 2
NS = 16
NW = NC * NS

GW = 128
N_FLAT = B * L
K = 5
KI = K * GW
PER_W = N_FLAT // NW
NCH = PER_W // KI


def _sc_gather(types_flat, table):
    mesh = plsc.VectorSubcoreMesh(core_axis_name="c", subcore_axis_name="s")

    @functools.partial(
        pl.kernel,
        mesh=mesh,
        out_type=jax.ShapeDtypeStruct((N_FLAT, 2 * CHANNELS), jnp.float32),
        scratch_types=[
            pltpu.VMEM((2, KI), jnp.int32),
            pltpu.VMEM((2, KI, CHANNELS), jnp.float32),
            pltpu.SemaphoreType.DMA,
            pltpu.SemaphoreType.DMA,
            pltpu.SemaphoreType.DMA,
            pltpu.SemaphoreType.DMA,
        ],
        compiler_params=pltpu.CompilerParams(use_tc_tiling_on_sc=False),
    )
    def run(idx_hbm, table_hbm, out_hbm, idx_v, rows_v,
            sem_g, sem_o0, sem_o1, sem_i):
        wid = lax.axis_index("s") * NC + lax.axis_index("c")
        w_base = wid * PER_W
        sems_o = (sem_o0, sem_o1)

        def prefetch_idx(c, buf):
            base = w_base + c * KI
            pltpu.async_copy(idx_hbm.at[pl.ds(base, KI)], idx_v.at[buf], sem_i)

        def wait_idx(c, buf):
            base = w_base + c * KI
            pltpu.make_async_copy(
                idx_hbm.at[pl.ds(base, KI)], idx_v.at[buf], sem_i
            ).wait()

        def fire_gathers(c, buf):
            for j in range(K):
                pltpu.async_copy(
                    table_hbm.at[idx_v.at[buf, pl.ds(j * GW, GW)]],
                    rows_v.at[buf, pl.ds(j * GW, GW)],
                    sem_g,
                )

        def drain_gathers(c, buf):
            for j in range(K):
                pltpu.make_async_copy(
                    table_hbm.at[idx_v.at[buf, pl.ds(j * GW, GW)]],
                    rows_v.at[buf, pl.ds(j * GW, GW)],
                    sem_g,
                ).wait()

        def out_copy(c, buf):
            base = w_base + c * KI
            pltpu.async_copy(
                rows_v.at[buf],
                out_hbm.at[pl.ds(base, KI), pl.ds(0, CHANNELS)],
                sems_o[buf],
            )

        def wait_out(c, buf):
            base = w_base + c * KI
            pltpu.make_async_copy(
                rows_v.at[buf],
                out_hbm.at[pl.ds(base, KI), pl.ds(0, CHANNELS)],
                sems_o[buf],
            ).wait()

        prefetch_idx(0, 0)
        prefetch_idx(1, 1)
        wait_idx(0, 0)
        fire_gathers(0, 0)

        def step(c, carry):
            def body_for(buf):
                nbuf = 1 - buf
                drain_gathers(c, buf)

                @pl.when(c + 1 < NCH)
                def _():
                    @pl.when(c >= 1)
                    def _():
                        wait_out(c - 1, nbuf)

                    wait_idx(c + 1, nbuf)
                    fire_gathers(c + 1, nbuf)

                out_copy(c, buf)

                @pl.when(c + 2 < NCH)
                def _():
                    prefetch_idx(c + 2, buf)

            lax.cond(c % 2 == 0, lambda: body_for(0), lambda: body_for(1))
            return carry

        lax.fori_loop(0, NCH, step, 0)
        wait_out(NCH - 2, (NCH - 2) % 2)
        wait_out(NCH - 1, (NCH - 1) % 2)

    return run(types_flat, table)


def kernel(types, table):
    out = _sc_gather(types.reshape(N_FLAT), table)
    return out[:, :CHANNELS].reshape(B, L, CHANNELS)
